# baseline (speedup 1.0000x reference)
import functools

import jax
import jax.numpy as jnp
from jax import lax
from jax.experimental import pallas as pl
from jax.experimental.pallas import tpu as pltpu
from jax.experimental.pallas import tpu_sc as plsc

NUM_NODES = 1000000
DIM = 64
BATCH = 16384

_INFO = plsc.get_sparse_core_info()
_NC = _INFO.num_cores
_NS = _INFO.num_subcores
_NW = _NC * _NS
_BPW = BATCH // _NW
_NSTREAM = _BPW // 128
_NCHUNK = 4
_CROWS = _BPW // _NCHUNK
_NGRP = _CROWS // 16

_PARAMS = pltpu.CompilerParams(
    needs_layout_passes=False, use_tc_tiling_on_sc=True)
_MESH = dict(core_axis_name="c", subcore_axis_name="s")


def _log_sigmoid(x):
    t = jnp.exp(-jnp.abs(x))
    z = t / (t + 2.0)
    z2 = z * z
    log1p = 2.0 * z * (1.0 + z2 * (1.0 / 3.0 + z2 * (0.2 + z2 * (1.0 / 7.0 + z2 * (1.0 / 9.0)))))
    return jnp.minimum(x, 0.0) - log1p


@functools.partial(
    pl.kernel,
    out_type=jax.ShapeDtypeStruct((BATCH, 128), jnp.float32),
    mesh=plsc.VectorSubcoreMesh(**_MESH),
    compiler_params=_PARAMS,
    scratch_types=[
        pltpu.VMEM((_NSTREAM, 128), jnp.int32),
        pltpu.VMEM((_NSTREAM, 128), jnp.int32),
        pltpu.VMEM((_BPW, 128), jnp.float32),
        pltpu.SemaphoreType.DMA,
    ],
)
def _gather(idx_hbm, tab_hbm, out_hbm, idx_v, pair_v, rows_v, sem):
    wid = lax.axis_index("s") * _NC + lax.axis_index("c")
    base = wid * _BPW

    for j in range(_NSTREAM):
        pltpu.sync_copy(idx_hbm.at[pl.ds(base + j * 128, 128)], idx_v.at[j])
    for j in range(_NSTREAM):
        for k in range(128 // 16):
            sl = pl.ds(k * 16, 16)
            pair_v[j, sl] = lax.shift_right_logical(idx_v[j, sl], 1)

    handles = []
    for j in range(_NSTREAM):
        handles.append(pltpu.async_copy(
            tab_hbm.at[pair_v.at[j]], rows_v.at[pl.ds(j * 128, 128)], sem))
    for h in handles:
        h.wait()
    pltpu.sync_copy(rows_v, out_hbm.at[pl.ds(base, _BPW)])


@functools.partial(
    pl.kernel,
    out_type=jax.ShapeDtypeStruct((_NW, 128), jnp.float32),
    mesh=plsc.VectorSubcoreMesh(**_MESH),
    compiler_params=_PARAMS,
    scratch_types=[
        pltpu.VMEM((_BPW,), jnp.int32),
        pltpu.VMEM((_BPW,), jnp.int32),
        pltpu.VMEM((_CROWS, 128), jnp.float32),
        pltpu.VMEM((_CROWS, 128), jnp.float32),
        pltpu.VMEM((_CROWS, 128), jnp.float32),
        pltpu.VMEM((_CROWS, 128), jnp.float32),
        pltpu.VMEM((_BPW,), jnp.float32),
        pltpu.VMEM((128,), jnp.float32),
        pltpu.SemaphoreType.DMA,
        pltpu.SemaphoreType.DMA,
    ],
)
def _loss(u_hbm, v_hbm, sign_hbm, e_hbm, c_hbm, out_hbm,
          u_idx, v_idx, ebuf0, ebuf1, cbuf0, cbuf1,
          sign_v, stage_v, sem0, sem1):
    wid = lax.axis_index("s") * _NC + lax.axis_index("c")
    base = wid * _BPW

    pltpu.sync_copy(u_hbm.at[pl.ds(base, _BPW)], u_idx)
    pltpu.sync_copy(v_hbm.at[pl.ds(base, _BPW)], v_idx)
    pltpu.sync_copy(sign_hbm.at[pl.ds(base, _BPW)], sign_v)

    bufs = [(ebuf0, cbuf0), (ebuf1, cbuf1)]
    sems = [sem0, sem1]
    handles = [None] * _NCHUNK

    def fire(c):
        eb, cb = bufs[c % 2]
        row0 = base + c * _CROWS
        handles[c] = (
            pltpu.async_copy(e_hbm.at[pl.ds(row0, _CROWS)], eb, sems[c % 2]),
            pltpu.async_copy(c_hbm.at[pl.ds(row0, _CROWS)], cb, sems[c % 2]),
        )

    fire(0)
    fire(1)

    lane = lax.iota(jnp.int32, 16)
    loss = jnp.zeros((16,), jnp.float32)
    for c in range(_NCHUNK):
        eb, cb = bufs[c % 2]
        he, hc = handles[c]
        he.wait()
        hc.wait()

        def group_body(g, acc_loss, c=c, eb=eb, cb=cb):
            rows = g * 16 + lane
            gsl = pl.ds(c * _CROWS + g * 16, 16)
            ucol = (u_idx[gsl] & 1) * DIM
            vcol = (v_idx[gsl] & 1) * DIM
            acc = jnp.zeros((16,), jnp.float32)
            for col in range(DIM):
                e = plsc.load_gather(eb, [rows, ucol + col])
                x = plsc.load_gather(cb, [rows, vcol + col])
                acc = acc + e * x
            x = acc * sign_v[gsl]
            return acc_loss + _log_sigmoid(x)

        loss = lax.fori_loop(0, _NGRP, group_body, loss)
        if c + 2 < _NCHUNK:
            fire(c + 2)

    zeros = jnp.zeros((16,), jnp.float32)
    for k in range(8):
        stage_v[pl.ds(k * 16, 16)] = loss if k == 0 else zeros
    pltpu.sync_copy(stage_v, out_hbm.at[wid])


def kernel(u, v, sign, emb, ctx):
    u = u.astype(jnp.int32)
    v = v.astype(jnp.int32)
    emb2 = emb.reshape(NUM_NODES // 2, 2 * DIM)
    ctx2 = ctx.reshape(NUM_NODES // 2, 2 * DIM)
    e_rows = _gather(u, emb2)
    c_rows = _gather(v, ctx2)
    partials = _loss(u, v, sign, e_rows, c_rows)
    return -jnp.sum(partials)


# speedup vs baseline: 2.4244x; 2.4244x over previous
import functools

import jax
import jax.numpy as jnp
from jax import lax
from jax.experimental import pallas as pl
from jax.experimental.pallas import tpu as pltpu
from jax.experimental.pallas import tpu_sc as plsc

NUM_NODES = 1000000
DIM = 64
BATCH = 16384

_INFO = plsc.get_sparse_core_info()
_NC = _INFO.num_cores
_NS = _INFO.num_subcores
_NW = _NC * _NS
_BPW = BATCH // _NW
_NGROUP = _BPW // 16


def _log_sigmoid(x):
    t = jnp.exp(-jnp.abs(x))
    z = t / (t + 2.0)
    z2 = z * z
    log1p = 2.0 * z * (1.0 + z2 * (1.0 / 3.0 + z2 * (0.2 + z2 * (1.0 / 7.0 + z2 * (1.0 / 9.0)))))
    return jnp.minimum(x, 0.0) - log1p


@functools.partial(
    pl.kernel,
    out_type=jax.ShapeDtypeStruct((_NW, 128), jnp.float32),
    mesh=plsc.VectorSubcoreMesh(core_axis_name="c", subcore_axis_name="s"),
    compiler_params=pltpu.CompilerParams(
        needs_layout_passes=False, use_tc_tiling_on_sc=True),
    scratch_types=[
        pltpu.VMEM((_BPW,), jnp.int32),
        pltpu.VMEM((_BPW,), jnp.int32),
        pltpu.VMEM((128, 128), jnp.float32),
        pltpu.VMEM((128, 128), jnp.float32),
        pltpu.VMEM((128, 128), jnp.float32),
        pltpu.VMEM((128, 128), jnp.float32),
        pltpu.VMEM((_BPW,), jnp.float32),
        pltpu.VMEM((128,), jnp.float32),
        pltpu.SemaphoreType.DMA,
        pltpu.SemaphoreType.DMA,
    ],
)
def _sc_loss(u_hbm, v_hbm, sign_hbm, embt_hbm, ctxt_hbm, out_hbm,
             u_idx, v_idx, eblk0, eblk1, cblk0, cblk1,
             sign_v, stage_v, sem0, sem1):
    wid = lax.axis_index("s") * _NC + lax.axis_index("c")
    base = wid * _BPW

    pltpu.sync_copy(u_hbm.at[pl.ds(base, _BPW)], u_idx)
    pltpu.sync_copy(v_hbm.at[pl.ds(base, _BPW)], v_idx)
    pltpu.sync_copy(sign_hbm.at[pl.ds(base, _BPW)], sign_v)

    ebufs = [eblk0, eblk1]
    cbufs = [cblk0, cblk1]
    sems = [sem0, sem1]
    lane = lax.iota(jnp.int32, 16)

    def copies(uu, vv, s, slot):
        out = []
        for j in (2 * s, 2 * s + 1):
            half = pl.ds((j % 2) * DIM, DIM)
            ub = pl.multiple_of((uu[j] >> 7) * 128, 128)
            vb = pl.multiple_of((vv[j] >> 7) * 128, 128)
            out.append((embt_hbm.at[:, pl.ds(ub, 128)],
                        ebufs[slot].at[half], sems[slot]))
            out.append((ctxt_hbm.at[:, pl.ds(vb, 128)],
                        cbufs[slot].at[half], sems[slot]))
        return out

    def fire(uu, vv, s, slot):
        for src, dst, sem in copies(uu, vv, s, slot):
            pltpu.async_copy(src, dst, sem)

    def wait(uu, vv, s, slot):
        for src, dst, sem in copies(uu, vv, s, slot):
            pltpu.make_async_copy(src, dst, sem).wait()

    def group_body(g, loss):
        gsl = pl.ds(g * 16, 16)
        uu = u_idx[gsl]
        vv = v_idx[gsl]
        fire(uu, vv, 0, 0)
        x16 = jnp.zeros((16,), jnp.float32)
        for s in range(8):
            slot = s % 2
            if s + 1 < 8:
                fire(uu, vv, s + 1, slot ^ 1)
            wait(uu, vv, s, slot)
            for j in (2 * s, 2 * s + 1):
                rbase = (j % 2) * DIM
                ucv = jnp.full((16,), uu[j] & 127, jnp.int32)
                vcv = jnp.full((16,), vv[j] & 127, jnp.int32)
                p = jnp.zeros((16,), jnp.float32)
                for c in range(DIM // 16):
                    rows = rbase + c * 16 + lane
                    e = plsc.load_gather(ebufs[slot], [rows, ucv])
                    x = plsc.load_gather(cbufs[slot], [rows, vcv])
                    p = p + e * x
                dot = jnp.sum(p)
                x16 = jnp.where(lane == j, dot, x16)
        x = x16 * sign_v[gsl]
        return loss + _log_sigmoid(x)

    loss = lax.fori_loop(0, _NGROUP, group_body, jnp.zeros((16,), jnp.float32))

    zeros = jnp.zeros((16,), jnp.float32)
    for k in range(8):
        stage_v[pl.ds(k * 16, 16)] = loss if k == 0 else zeros
    pltpu.sync_copy(stage_v, out_hbm.at[wid])


def kernel(u, v, sign, emb, ctx):
    partials = _sc_loss(u.astype(jnp.int32), v.astype(jnp.int32),
                        sign, emb.T, ctx.T)
    return -jnp.sum(partials)


# speedup vs baseline: 2.5076x; 1.0343x over previous
import functools

import jax
import jax.numpy as jnp
from jax import lax
from jax.experimental import pallas as pl
from jax.experimental.pallas import tpu as pltpu
from jax.experimental.pallas import tpu_sc as plsc

NUM_NODES = 1000000
DIM = 64
BATCH = 16384

_INFO = plsc.get_sparse_core_info()
_NC = _INFO.num_cores
_NS = _INFO.num_subcores
_NW = _NC * _NS
_BPW = BATCH // _NW
_NGROUP = _BPW // 16
_NSLOT = 4


def _log_sigmoid(x):
    t = jnp.exp(-jnp.abs(x))
    z = t / (t + 2.0)
    z2 = z * z
    log1p = 2.0 * z * (1.0 + z2 * (1.0 / 3.0 + z2 * (0.2 + z2 * (1.0 / 7.0 + z2 * (1.0 / 9.0)))))
    return jnp.minimum(x, 0.0) - log1p


@functools.partial(
    pl.kernel,
    out_type=jax.ShapeDtypeStruct((_NW, 128), jnp.float32),
    mesh=plsc.VectorSubcoreMesh(core_axis_name="c", subcore_axis_name="s"),
    compiler_params=pltpu.CompilerParams(
        needs_layout_passes=False, use_tc_tiling_on_sc=True),
    scratch_types=(
        [pltpu.VMEM((_BPW,), jnp.int32)] * 2 +
        [pltpu.VMEM((DIM, 128), jnp.float32)] * 8 +
        [pltpu.VMEM((_BPW,), jnp.float32),
         pltpu.VMEM((128,), jnp.float32)] +
        [pltpu.SemaphoreType.DMA] * _NSLOT
    ),
)
def _sc_loss(u_hbm, v_hbm, sign_hbm, embt_hbm, ctxt_hbm, out_hbm,
             u_idx, v_idx,
             eb0, eb1, eb2, eb3, cb0, cb1, cb2, cb3,
             sign_v, stage_v, sem0, sem1, sem2, sem3):
    wid = lax.axis_index("s") * _NC + lax.axis_index("c")
    base = wid * _BPW

    pltpu.sync_copy(u_hbm.at[pl.ds(base, _BPW)], u_idx)
    pltpu.sync_copy(v_hbm.at[pl.ds(base, _BPW)], v_idx)
    pltpu.sync_copy(sign_hbm.at[pl.ds(base, _BPW)], sign_v)

    ebufs = [eb0, eb1, eb2, eb3]
    cbufs = [cb0, cb1, cb2, cb3]
    sems = [sem0, sem1, sem2, sem3]
    lane = lax.iota(jnp.int32, 16)

    def copies(uu, vv, j, slot):
        ub = pl.multiple_of((uu[j] >> 7) * 128, 128)
        vb = pl.multiple_of((vv[j] >> 7) * 128, 128)
        return ((embt_hbm.at[:, pl.ds(ub, 128)], ebufs[slot], sems[slot]),
                (ctxt_hbm.at[:, pl.ds(vb, 128)], cbufs[slot], sems[slot]))

    def fire(uu, vv, j, slot):
        for src, dst, sem in copies(uu, vv, j, slot):
            pltpu.async_copy(src, dst, sem)

    def wait(uu, vv, j, slot):
        for src, dst, sem in copies(uu, vv, j, slot):
            pltpu.make_async_copy(src, dst, sem).wait()

    uu0 = u_idx[pl.ds(0, 16)]
    vv0 = v_idx[pl.ds(0, 16)]
    fire(uu0, vv0, 0, 0)
    fire(uu0, vv0, 1, 1)

    def group_body(g, loss):
        gsl = pl.ds(g * 16, 16)
        uu = u_idx[gsl]
        vv = v_idx[gsl]
        gn = jnp.minimum(g + 1, _NGROUP - 1)
        nsl = pl.ds(gn * 16, 16)
        uun = u_idx[nsl]
        vvn = v_idx[nsl]

        x16 = jnp.zeros((16,), jnp.float32)
        for s in range(16):
            slot = s % _NSLOT
            fslot = (s + 2) % _NSLOT
            if s + 2 < 16:
                fire(uu, vv, s + 2, fslot)
            else:
                fire(uun, vvn, s + 2 - 16, fslot)
            wait(uu, vv, s, slot)
            ucv = jnp.full((16,), uu[s] & 127, jnp.int32)
            vcv = jnp.full((16,), vv[s] & 127, jnp.int32)
            p = jnp.zeros((16,), jnp.float32)
            for c in range(DIM // 16):
                rows = c * 16 + lane
                e = plsc.load_gather(ebufs[slot], [rows, ucv])
                x = plsc.load_gather(cbufs[slot], [rows, vcv])
                p = p + e * x
            dot = jnp.sum(p)
            x16 = jnp.where(lane == s, dot, x16)
        x = x16 * sign_v[gsl]
        return loss + _log_sigmoid(x)

    loss = lax.fori_loop(0, _NGROUP, group_body, jnp.zeros((16,), jnp.float32))

    uuL = u_idx[pl.ds((_NGROUP - 1) * 16, 16)]
    vvL = v_idx[pl.ds((_NGROUP - 1) * 16, 16)]
    wait(uuL, vvL, 0, 0)
    wait(uuL, vvL, 1, 1)

    zeros = jnp.zeros((16,), jnp.float32)
    for k in range(8):
        stage_v[pl.ds(k * 16, 16)] = loss if k == 0 else zeros
    pltpu.sync_copy(stage_v, out_hbm.at[wid])


def kernel(u, v, sign, emb, ctx):
    partials = _sc_loss(u.astype(jnp.int32), v.astype(jnp.int32),
                        sign, emb.T, ctx.T)
    return -jnp.sum(partials)


# speedup vs baseline: 2.7285x; 1.0881x over previous
import functools

import jax
import jax.numpy as jnp
from jax import lax
from jax.experimental import pallas as pl
from jax.experimental.pallas import tpu as pltpu
from jax.experimental.pallas import tpu_sc as plsc

NUM_NODES = 1000000
DIM = 64
BATCH = 16384

_INFO = plsc.get_sparse_core_info()
_NC = _INFO.num_cores
_NS = _INFO.num_subcores
_NW = _NC * _NS
_BPW = BATCH // _NW
_NGROUP = _BPW // 16
_NSLOT = 4


def _log_sigmoid(x):
    t = jnp.exp(-jnp.abs(x))
    z = t / (t + 2.0)
    z2 = z * z
    log1p = 2.0 * z * (1.0 + z2 * (1.0 / 3.0 + z2 * (0.2 + z2 * (1.0 / 7.0 + z2 * (1.0 / 9.0)))))
    return jnp.minimum(x, 0.0) - log1p


@functools.partial(
    pl.kernel,
    out_type=jax.ShapeDtypeStruct((_NW, 128), jnp.float32),
    mesh=plsc.VectorSubcoreMesh(core_axis_name="c", subcore_axis_name="s"),
    compiler_params=pltpu.CompilerParams(
        needs_layout_passes=False, use_tc_tiling_on_sc=True),
    scratch_types=(
        [pltpu.VMEM((_BPW,), jnp.int32)] * 2 +
        [pltpu.VMEM((DIM, 128), jnp.float32)] * 8 +
        [pltpu.VMEM((_BPW,), jnp.float32),
         pltpu.VMEM((128,), jnp.float32)] +
        [pltpu.SemaphoreType.DMA] * _NSLOT
    ),
)
def _sc_loss(u_hbm, v_hbm, sign_hbm, embt_hbm, ctxt_hbm, out_hbm,
             u_idx, v_idx,
             eb0, eb1, eb2, eb3, cb0, cb1, cb2, cb3,
             sign_v, stage_v, sem0, sem1, sem2, sem3):
    wid = lax.axis_index("s") * _NC + lax.axis_index("c")
    base = wid * _BPW

    pltpu.sync_copy(u_hbm.at[pl.ds(base, _BPW)], u_idx)
    pltpu.sync_copy(v_hbm.at[pl.ds(base, _BPW)], v_idx)
    pltpu.sync_copy(sign_hbm.at[pl.ds(base, _BPW)], sign_v)

    ebufs = [eb0, eb1, eb2, eb3]
    cbufs = [cb0, cb1, cb2, cb3]
    sems = [sem0, sem1, sem2, sem3]
    lane = lax.iota(jnp.int32, 16)

    def copies(uu, vv, j, slot):
        ub = pl.multiple_of((uu[j] >> 7) * 128, 128)
        vb = pl.multiple_of((vv[j] >> 7) * 128, 128)
        return ((embt_hbm.at[:, pl.ds(ub, 128)], ebufs[slot], sems[slot]),
                (ctxt_hbm.at[:, pl.ds(vb, 128)], cbufs[slot], sems[slot]))

    def fire(uu, vv, j, slot):
        for src, dst, sem in copies(uu, vv, j, slot):
            pltpu.async_copy(src, dst, sem)

    def wait(uu, vv, j, slot):
        for src, dst, sem in copies(uu, vv, j, slot):
            pltpu.make_async_copy(src, dst, sem).wait()

    uu0 = u_idx[pl.ds(0, 16)]
    vv0 = v_idx[pl.ds(0, 16)]
    fire(uu0, vv0, 0, 0)
    fire(uu0, vv0, 1, 1)
    fire(uu0, vv0, 2, 2)

    def group_body(g, loss):
        gsl = pl.ds(g * 16, 16)
        uu = u_idx[gsl]
        vv = v_idx[gsl]
        gn = jnp.minimum(g + 1, _NGROUP - 1)
        nsl = pl.ds(gn * 16, 16)
        uun = u_idx[nsl]
        vvn = v_idx[nsl]

        x16 = jnp.zeros((16,), jnp.float32)
        for s in range(16):
            slot = s % _NSLOT
            fslot = (s + 3) % _NSLOT
            if s + 3 < 16:
                fire(uu, vv, s + 3, fslot)
            else:
                fire(uun, vvn, s + 3 - 16, fslot)
            wait(uu, vv, s, slot)
            ucv = jnp.full((16,), uu[s] & 127, jnp.int32)
            vcv = jnp.full((16,), vv[s] & 127, jnp.int32)
            p = jnp.zeros((16,), jnp.float32)
            for c in range(DIM // 16):
                rows = c * 16 + lane
                e = plsc.load_gather(ebufs[slot], [rows, ucv])
                x = plsc.load_gather(cbufs[slot], [rows, vcv])
                p = p + e * x
            dot = jnp.sum(p)
            x16 = jnp.where(lane == s, dot, x16)
        x = x16 * sign_v[gsl]
        return loss + _log_sigmoid(x)

    loss = lax.fori_loop(0, _NGROUP, group_body, jnp.zeros((16,), jnp.float32))

    uuL = u_idx[pl.ds((_NGROUP - 1) * 16, 16)]
    vvL = v_idx[pl.ds((_NGROUP - 1) * 16, 16)]
    wait(uuL, vvL, 0, 0)
    wait(uuL, vvL, 1, 1)
    wait(uuL, vvL, 2, 2)

    zeros = jnp.zeros((16,), jnp.float32)
    for k in range(8):
        stage_v[pl.ds(k * 16, 16)] = loss if k == 0 else zeros
    pltpu.sync_copy(stage_v, out_hbm.at[wid])


def kernel(u, v, sign, emb, ctx):
    partials = _sc_loss(u.astype(jnp.int32), v.astype(jnp.int32),
                        sign, emb.T, ctx.T)
    return -jnp.sum(partials)
